# Initial kernel scaffold; baseline (speedup 1.0000x reference)
import jax
import jax.numpy as jnp
from jax.experimental import pallas as pl


def kernel(pcs):
    raise NotImplementedError("write your pallas kernel here")


# speedup vs baseline: 3.7713x; 3.7713x over previous
import functools

import jax
import jax.numpy as jnp
from jax.experimental import pallas as pl

_RADIUS = 0.2
_MAX_SAMPLES = 64
_BI = 256


def _bq_kernel(pcs_ref, out_ref):
    i = pl.program_id(1)
    xall = pcs_ref[0]
    n = xall.shape[1]
    xblk = pcs_ref[0, :, pl.ds(i * _BI, _BI)]

    sq_all = xall[0] * xall[0] + xall[1] * xall[1] + xall[2] * xall[2]
    sq_blk = xblk[0] * xblk[0] + xblk[1] * xblk[1] + xblk[2] * xblk[2]
    dot = jnp.dot(xblk.T, xall, preferred_element_type=jnp.float32)
    d2 = (sq_blk[:, None] + sq_all[None, :]) - 2.0 * dot
    mask = (d2 < _RADIUS * _RADIUS).astype(jnp.int32)

    c = mask
    k = 1
    while k < n:
        c = c + jnp.concatenate(
            [jnp.zeros((_BI, k), jnp.int32), c[:, : n - k]], axis=1)
        k *= 2
    cnt = c[:, n - 1][:, None]

    cols = [jnp.sum((c <= s).astype(jnp.int32), axis=1, keepdims=True)
            for s in range(_MAX_SAMPLES)]
    sel = jnp.concatenate(cols, axis=1)

    pos = jax.lax.broadcasted_iota(jnp.int32, (_BI, _MAX_SAMPLES), 1)
    res = jnp.where(pos < cnt, sel, sel[:, 0][:, None])
    out_ref[0] = res


@jax.jit
def kernel(pcs):
    b, _, n = pcs.shape
    out = pl.pallas_call(
        _bq_kernel,
        grid=(b, n // _BI),
        in_specs=[pl.BlockSpec((1, 3, n), lambda bb, ii: (bb, 0, 0))],
        out_specs=pl.BlockSpec((1, _BI, _MAX_SAMPLES), lambda bb, ii: (bb, ii, 0)),
        out_shape=jax.ShapeDtypeStruct((b, n, _MAX_SAMPLES), jnp.int32),
    )(pcs)
    return out.astype(jnp.int64)


# speedup vs baseline: 4.8878x; 1.2961x over previous
import functools

import jax
import jax.numpy as jnp
from jax import lax
from jax.experimental import pallas as pl
from jax.experimental.pallas import tpu as pltpu
from jax.experimental.pallas import tpu_sc as plsc

_RADIUS = 0.2
_MAX_SAMPLES = 64
_BI = 256
_NC = 2
_NS = 16
_CR = 16


def _rank_tc_kernel(pcs_ref, g_ref, cnt_ref):
    i = pl.program_id(1)
    xall = pcs_ref[0]
    n = xall.shape[1]
    xblk = pcs_ref[0, :, pl.ds(i * _BI, _BI)]

    sq_all = xall[0] * xall[0] + xall[1] * xall[1] + xall[2] * xall[2]
    sq_blk = xblk[0] * xblk[0] + xblk[1] * xblk[1] + xblk[2] * xblk[2]
    dot = jnp.dot(xblk.T, xall, preferred_element_type=jnp.float32)
    d2 = (sq_blk[:, None] + sq_all[None, :]) - 2.0 * dot
    mask = d2 < _RADIUS * _RADIUS

    c = mask.astype(jnp.int32)
    k = 1
    while k < n:
        c = c + jnp.concatenate(
            [jnp.zeros((_BI, k), jnp.int32), c[:, : n - k]], axis=1)
        k *= 2

    g = jnp.where(mask & (c <= _MAX_SAMPLES), c, 0)
    g_ref[0] = g
    cnt_ref[0] = c[:, n - 1:n]


def _sc_scatter_kernel(g_hbm, cnt_hbm, out_hbm, buf, cntbuf, outbuf):
    b_per_batch = 4
    rows_per_worker = 512
    n_chunks = rows_per_worker // _CR
    wid = lax.axis_index("s") * _NC + lax.axis_index("c")
    batch = wid // b_per_batch
    lr0 = (wid % b_per_batch) * rows_per_worker

    iota = lax.broadcasted_iota(jnp.int32, (16,), 0)
    zeros16 = jnp.zeros((16,), jnp.int32)

    def chunk_body(ci, _):
        r0 = lr0 + ci * _CR
        pltpu.sync_copy(g_hbm.at[batch, pl.ds(r0, _CR)], buf)
        pltpu.sync_copy(cnt_hbm.at[batch, pl.ds(r0, _CR)], cntbuf)

        def row_body(r, _):
            rsplat = jnp.full((16,), r, jnp.int32)

            def vec_body(k, jv):
                v = buf[r, pl.ds(k * 16, 16)]
                m = v > 0
                plsc.store_scatter(outbuf, [rsplat, v - 1], jv, mask=m)
                return jv + 16

            jv0 = iota
            lax.fori_loop(0, 2048 // 16, vec_body, jv0)

            cntv = plsc.load_gather(cntbuf, [rsplat, zeros16])
            firstv = plsc.load_gather(outbuf, [rsplat, zeros16])
            for t in range(_MAX_SAMPLES // 16):
                sv = iota + (t * 16)
                cur = outbuf[r, pl.ds(t * 16, 16)]
                outbuf[r, pl.ds(t * 16, 16)] = jnp.where(sv < cntv, cur, firstv)
            return 0

        lax.fori_loop(0, _CR, row_body, 0)
        pltpu.sync_copy(outbuf, out_hbm.at[batch, pl.ds(r0, _CR)])
        return 0

    lax.fori_loop(0, n_chunks, chunk_body, 0)


@jax.jit
def kernel(pcs):
    b, _, n = pcs.shape
    g, cnt = pl.pallas_call(
        _rank_tc_kernel,
        grid=(b, n // _BI),
        in_specs=[pl.BlockSpec((1, 3, n), lambda bb, ii: (bb, 0, 0))],
        out_specs=[
            pl.BlockSpec((1, _BI, n), lambda bb, ii: (bb, ii, 0)),
            pl.BlockSpec((1, _BI, 1), lambda bb, ii: (bb, ii, 0)),
        ],
        out_shape=[
            jax.ShapeDtypeStruct((b, n, n), jnp.int32),
            jax.ShapeDtypeStruct((b, n, 1), jnp.int32),
        ],
    )(pcs)

    mesh = plsc.VectorSubcoreMesh(
        core_axis_name="c", subcore_axis_name="s",
        num_cores=_NC, num_subcores=_NS)
    sc = pl.kernel(
        _sc_scatter_kernel,
        out_type=jax.ShapeDtypeStruct((b, n, _MAX_SAMPLES), jnp.int32),
        mesh=mesh,
        scratch_types=[
            pltpu.VMEM((_CR, n), jnp.int32),
            pltpu.VMEM((_CR, 1), jnp.int32),
            pltpu.VMEM((_CR, _MAX_SAMPLES), jnp.int32),
        ],
        compiler_params=pltpu.CompilerParams(needs_layout_passes=False),
    )
    out = sc(g, cnt)
    return out.astype(jnp.int64)


# speedup vs baseline: 5.0555x; 1.0343x over previous
import functools

import jax
import jax.numpy as jnp
from jax import lax
from jax.experimental import pallas as pl
from jax.experimental.pallas import tpu as pltpu
from jax.experimental.pallas import tpu_sc as plsc

_RADIUS = 0.2
_MAX_SAMPLES = 64
_BI = 256
_NC = 2
_NS = 16
_CR = 16


def _rank_tc_kernel(pcs_ref, g_ref, cnt_ref):
    i = pl.program_id(1)
    xall = pcs_ref[0]
    n = xall.shape[1]
    xblk = pcs_ref[0, :, pl.ds(i * _BI, _BI)]

    sq_all = xall[0] * xall[0] + xall[1] * xall[1] + xall[2] * xall[2]
    sq_blk = xblk[0] * xblk[0] + xblk[1] * xblk[1] + xblk[2] * xblk[2]
    dot = jnp.dot(xblk.T, xall, preferred_element_type=jnp.float32)
    d2 = (sq_blk[:, None] + sq_all[None, :]) - 2.0 * dot
    mask = d2 < _RADIUS * _RADIUS

    c = mask.astype(jnp.int32)
    k = 1
    while k < n:
        c = c + jnp.concatenate(
            [jnp.zeros((_BI, k), jnp.int32), c[:, : n - k]], axis=1)
        k *= 2

    g = jnp.where(mask & (c <= _MAX_SAMPLES), c, 0)
    g_ref[0] = g
    cnt_ref[0] = c[:, n - 1:n]


def _sc_scatter_kernel(g_hbm, cnt_hbm, out_hbm, buf, cntbuf, outbuf):
    b_per_batch = 4
    rows_per_worker = 512
    n_chunks = rows_per_worker // _CR
    wid = lax.axis_index("s") * _NC + lax.axis_index("c")
    batch = wid // b_per_batch
    lr0 = (wid % b_per_batch) * rows_per_worker

    iota = lax.broadcasted_iota(jnp.int32, (16,), 0)
    zeros16 = jnp.zeros((16,), jnp.int32)

    def chunk_body(ci, _):
        r0 = lr0 + ci * _CR
        pltpu.sync_copy(g_hbm.at[batch, pl.ds(r0, _CR)], buf)
        pltpu.sync_copy(cnt_hbm.at[batch, pl.ds(r0, _CR)], cntbuf)

        def row_body(r, _):
            rsplat = jnp.full((16,), r, jnp.int32)

            def vec_body(k, jv):
                v = buf[r, pl.ds(k * 16, 16)]
                m = v > 0
                plsc.store_scatter(outbuf, [rsplat, v - 1], jv, mask=m)
                return jv + 16

            jv0 = iota
            lax.fori_loop(0, 2048 // 16, vec_body, jv0, unroll=8)

            cntv = plsc.load_gather(cntbuf, [rsplat, zeros16])
            firstv = plsc.load_gather(outbuf, [rsplat, zeros16])
            for t in range(_MAX_SAMPLES // 16):
                sv = iota + (t * 16)
                cur = outbuf[r, pl.ds(t * 16, 16)]
                outbuf[r, pl.ds(t * 16, 16)] = jnp.where(sv < cntv, cur, firstv)
            return 0

        lax.fori_loop(0, _CR, row_body, 0)
        pltpu.sync_copy(outbuf, out_hbm.at[batch, pl.ds(r0, _CR)])
        return 0

    lax.fori_loop(0, n_chunks, chunk_body, 0)


@jax.jit
def kernel(pcs):
    b, _, n = pcs.shape
    g, cnt = pl.pallas_call(
        _rank_tc_kernel,
        grid=(b, n // _BI),
        in_specs=[pl.BlockSpec((1, 3, n), lambda bb, ii: (bb, 0, 0))],
        out_specs=[
            pl.BlockSpec((1, _BI, n), lambda bb, ii: (bb, ii, 0)),
            pl.BlockSpec((1, _BI, 1), lambda bb, ii: (bb, ii, 0)),
        ],
        out_shape=[
            jax.ShapeDtypeStruct((b, n, n), jnp.int32),
            jax.ShapeDtypeStruct((b, n, 1), jnp.int32),
        ],
    )(pcs)

    mesh = plsc.VectorSubcoreMesh(
        core_axis_name="c", subcore_axis_name="s",
        num_cores=_NC, num_subcores=_NS)
    sc = pl.kernel(
        _sc_scatter_kernel,
        out_type=jax.ShapeDtypeStruct((b, n, _MAX_SAMPLES), jnp.int32),
        mesh=mesh,
        scratch_types=[
            pltpu.VMEM((_CR, n), jnp.int32),
            pltpu.VMEM((_CR, 1), jnp.int32),
            pltpu.VMEM((_CR, _MAX_SAMPLES), jnp.int32),
        ],
        compiler_params=pltpu.CompilerParams(needs_layout_passes=False),
    )
    out = sc(g, cnt)
    return out.astype(jnp.int64)


# speedup vs baseline: 9.5093x; 1.8810x over previous
import functools

import jax
import jax.numpy as jnp
from jax import lax
from jax.experimental import pallas as pl
from jax.experimental.pallas import tpu as pltpu
from jax.experimental.pallas import tpu_sc as plsc

_RADIUS = 0.2
_MAX_SAMPLES = 64
_BI = 256
_NC = 2
_NS = 16
_CR = 16


def _rank_tc_kernel(pcs_ref, g_ref, cnt_ref):
    i = pl.program_id(1)
    xall = pcs_ref[0]
    n = xall.shape[1]
    xblk = pcs_ref[0, :, pl.ds(i * _BI, _BI)]

    sq_all = xall[0] * xall[0] + xall[1] * xall[1] + xall[2] * xall[2]
    sq_blk = xblk[0] * xblk[0] + xblk[1] * xblk[1] + xblk[2] * xblk[2]
    dot = jnp.dot(xblk.T, xall, preferred_element_type=jnp.float32)
    d2 = (sq_blk[:, None] + sq_all[None, :]) - 2.0 * dot
    mask = d2 < _RADIUS * _RADIUS

    c = mask.astype(jnp.int32)
    k = 1
    while k < n:
        c = c + jnp.concatenate(
            [jnp.zeros((_BI, k), jnp.int32), c[:, : n - k]], axis=1)
        k *= 2

    g = jnp.where(mask & (c <= _MAX_SAMPLES), c, 0)
    g_ref[0] = g
    cnt_ref[0] = c[:, n - 1:n]


def _sc_scatter_kernel(g_hbm, cnt_hbm, out_hbm, buf, cntbuf, outbuf):
    b_per_batch = 4
    rows_per_worker = 512
    n_chunks = rows_per_worker // _CR
    wid = lax.axis_index("s") * _NC + lax.axis_index("c")
    batch = wid // b_per_batch
    lr0 = (wid % b_per_batch) * rows_per_worker

    iota = lax.broadcasted_iota(jnp.int32, (16,), 0)
    zeros16 = jnp.zeros((16,), jnp.int32)

    def chunk_body(ci, _):
        r0 = lr0 + ci * _CR
        pltpu.sync_copy(g_hbm.at[batch, pl.ds(r0, _CR)], buf)
        pltpu.sync_copy(cnt_hbm.at[batch, pl.ds(r0, _CR)], cntbuf)

        def row_body(r, _):
            rsplat = jnp.full((16,), r, jnp.int32)

            @plsc.parallel_loop(0, 2048 // 16, unroll=8)
            def _vec_body(k):
                v = buf[r, pl.ds(k * 16, 16)]
                m = v > 0
                jv = iota + k * 16
                plsc.store_scatter(outbuf, [rsplat, v - 1], jv, mask=m)

            cntv = plsc.load_gather(cntbuf, [rsplat, zeros16])
            firstv = plsc.load_gather(outbuf, [rsplat, zeros16])
            for t in range(_MAX_SAMPLES // 16):
                sv = iota + (t * 16)
                cur = outbuf[r, pl.ds(t * 16, 16)]
                outbuf[r, pl.ds(t * 16, 16)] = jnp.where(sv < cntv, cur, firstv)
            return 0

        lax.fori_loop(0, _CR, row_body, 0)
        pltpu.sync_copy(outbuf, out_hbm.at[batch, pl.ds(r0, _CR)])
        return 0

    lax.fori_loop(0, n_chunks, chunk_body, 0)


@jax.jit
def kernel(pcs):
    b, _, n = pcs.shape
    g, cnt = pl.pallas_call(
        _rank_tc_kernel,
        grid=(b, n // _BI),
        in_specs=[pl.BlockSpec((1, 3, n), lambda bb, ii: (bb, 0, 0))],
        out_specs=[
            pl.BlockSpec((1, _BI, n), lambda bb, ii: (bb, ii, 0)),
            pl.BlockSpec((1, _BI, 1), lambda bb, ii: (bb, ii, 0)),
        ],
        out_shape=[
            jax.ShapeDtypeStruct((b, n, n), jnp.int32),
            jax.ShapeDtypeStruct((b, n, 1), jnp.int32),
        ],
    )(pcs)

    mesh = plsc.VectorSubcoreMesh(
        core_axis_name="c", subcore_axis_name="s",
        num_cores=_NC, num_subcores=_NS)
    sc = pl.kernel(
        _sc_scatter_kernel,
        out_type=jax.ShapeDtypeStruct((b, n, _MAX_SAMPLES), jnp.int32),
        mesh=mesh,
        scratch_types=[
            pltpu.VMEM((_CR, n), jnp.int32),
            pltpu.VMEM((_CR, 1), jnp.int32),
            pltpu.VMEM((_CR, _MAX_SAMPLES), jnp.int32),
        ],
        compiler_params=pltpu.CompilerParams(needs_layout_passes=False),
    )
    out = sc(g, cnt)
    return out.astype(jnp.int64)


# speedup vs baseline: 12.1338x; 1.2760x over previous
import functools

import jax
import jax.numpy as jnp
from jax import lax
from jax.experimental import pallas as pl
from jax.experimental.pallas import tpu as pltpu
from jax.experimental.pallas import tpu_sc as plsc

_RADIUS = 0.2
_MAX_SAMPLES = 64
_BI = 256
_NC = 2
_NS = 16
_CR = 16


def _rank_tc_kernel(pcs_ref, g_ref, cnt_ref):
    i = pl.program_id(1)
    xall = pcs_ref[0]
    n = xall.shape[1]
    xblk = pcs_ref[0, :, pl.ds(i * _BI, _BI)]

    sq_all = xall[0] * xall[0] + xall[1] * xall[1] + xall[2] * xall[2]
    sq_blk = xblk[0] * xblk[0] + xblk[1] * xblk[1] + xblk[2] * xblk[2]
    dot = jnp.dot(xblk.T, xall, preferred_element_type=jnp.float32)
    d2 = (sq_blk[:, None] + sq_all[None, :]) - 2.0 * dot
    mask = d2 < _RADIUS * _RADIUS

    c = mask.astype(jnp.int16)
    k = 1
    while k < n:
        c = c + jnp.concatenate(
            [jnp.zeros((_BI, k), jnp.int16), c[:, : n - k]], axis=1)
        k *= 2

    g = jnp.where(mask & (c <= _MAX_SAMPLES), c, jnp.int16(0))
    g_ref[0] = g
    cnt_ref[0] = c[:, n - 1:n].astype(jnp.int32)


def _sc_scatter_kernel(g_hbm, cnt_hbm, out_hbm, buf, cntbuf, outbuf):
    b_per_batch = 4
    rows_per_worker = 512
    n_chunks = rows_per_worker // _CR
    wid = lax.axis_index("s") * _NC + lax.axis_index("c")
    batch = wid // b_per_batch
    lr0 = (wid % b_per_batch) * rows_per_worker

    iota = lax.broadcasted_iota(jnp.int32, (16,), 0)
    iota2 = iota * 2
    zeros16 = jnp.zeros((16,), jnp.int32)

    def chunk_body(ci, _):
        r0 = lr0 + ci * _CR
        pltpu.sync_copy(g_hbm.at[batch, pl.ds(r0, _CR)], buf)
        pltpu.sync_copy(cnt_hbm.at[batch, pl.ds(r0, _CR)], cntbuf)

        def row_body(r, _):
            rsplat = jnp.full((16,), r, jnp.int32)

            @plsc.parallel_loop(0, 2048 // 32, unroll=8)
            def _vec_body(k):
                v16 = buf[r, pl.ds(k * 32, 32)]
                a, b = plsc.unpack(v16, format=plsc.PackFormat.INTERLEAVED)
                jv = iota2 + k * 32
                plsc.store_scatter(outbuf, [rsplat, a - 1], jv, mask=a > 0)
                plsc.store_scatter(outbuf, [rsplat, b - 1], jv + 1, mask=b > 0)

            cntv = plsc.load_gather(cntbuf, [rsplat, zeros16])
            firstv = plsc.load_gather(outbuf, [rsplat, zeros16])
            for t in range(_MAX_SAMPLES // 16):
                sv = iota + (t * 16)
                cur = outbuf[r, pl.ds(t * 16, 16)]
                outbuf[r, pl.ds(t * 16, 16)] = jnp.where(sv < cntv, cur, firstv)
            return 0

        lax.fori_loop(0, _CR, row_body, 0)
        pltpu.sync_copy(outbuf, out_hbm.at[batch, pl.ds(r0, _CR)])
        return 0

    lax.fori_loop(0, n_chunks, chunk_body, 0)


@jax.jit
def kernel(pcs):
    b, _, n = pcs.shape
    g, cnt = pl.pallas_call(
        _rank_tc_kernel,
        grid=(b, n // _BI),
        in_specs=[pl.BlockSpec((1, 3, n), lambda bb, ii: (bb, 0, 0))],
        out_specs=[
            pl.BlockSpec((1, _BI, n), lambda bb, ii: (bb, ii, 0)),
            pl.BlockSpec((1, _BI, 1), lambda bb, ii: (bb, ii, 0)),
        ],
        out_shape=[
            jax.ShapeDtypeStruct((b, n, n), jnp.int16),
            jax.ShapeDtypeStruct((b, n, 1), jnp.int32),
        ],
    )(pcs)

    mesh = plsc.VectorSubcoreMesh(
        core_axis_name="c", subcore_axis_name="s",
        num_cores=_NC, num_subcores=_NS)
    sc = pl.kernel(
        _sc_scatter_kernel,
        out_type=jax.ShapeDtypeStruct((b, n, _MAX_SAMPLES), jnp.int32),
        mesh=mesh,
        scratch_types=[
            pltpu.VMEM((_CR, n), jnp.int16),
            pltpu.VMEM((_CR, 1), jnp.int32),
            pltpu.VMEM((_CR, _MAX_SAMPLES), jnp.int32),
        ],
        compiler_params=pltpu.CompilerParams(needs_layout_passes=False),
    )
    out = sc(g, cnt)
    return out.astype(jnp.int64)
